# baseline (speedup 1.0000x reference)
import functools

import jax
import jax.numpy as jnp
from jax import lax
from jax.experimental import pallas as pl
from jax.experimental.pallas import tpu as pltpu
from jax.experimental.pallas import tpu_sc as plsc


def _conv_body(x0_ref, ep_ref, en_ref, wt_ref, bc_ref, pxt_ref,
               feat_ref, seg_ref, sct_ref, nrm_ref):
    tile = x0_ref.shape[1]
    x0 = x0_ref[0].astype(jnp.bfloat16)
    xm = jnp.concatenate([ep_ref[0, 0].astype(jnp.bfloat16), x0[:tile - 1, :]],
                         axis=0)
    xp = jnp.concatenate([x0[1:, :], en_ref[0, 0].astype(jnp.bfloat16)], axis=0)
    acc = jnp.dot(xm, wt_ref[0], preferred_element_type=jnp.float32)
    acc += jnp.dot(x0, wt_ref[1], preferred_element_type=jnp.float32)
    acc += jnp.dot(xp, wt_ref[2], preferred_element_type=jnp.float32)
    feat = jnp.maximum(acc + bc_ref[...], 0.0)
    feat_ref[0] = feat
    seg = jnp.dot(feat, pxt_ref[...], preferred_element_type=jnp.float32)
    seg_ref[0] = seg
    sct_ref[0] = seg.T
    nrm_ref[0, 0] = jnp.sqrt(jnp.sum(feat * feat, axis=1))


_HI = 0x80000000


def _sum16(v):
    tot = v[0]
    for i in range(1, 16):
        tot = tot + v[i]
    return tot


def _sc_topk_body(k, bc_total, ncls, t_len,
                  sct_ref, nrm_ref, out_ref, srow, nrow, skey, obuf):
    nchunk = t_len // 16
    wid = lax.axis_index("s") * 2 + lax.axis_index("c")
    for j in range((bc_total + 31) // 32):
        r = wid + 32 * j

        @pl.when(r < bc_total)
        def _slot():
            b = r // ncls
            pltpu.sync_copy(sct_ref.at[r], srow)
            pltpu.sync_copy(nrm_ref.at[b], nrow)

            def mk(cc, carry):
                bu = lax.bitcast_convert_type(srow[pl.ds(cc * 16, 16)],
                                              jnp.uint32)
                neg = bu >= jnp.uint32(_HI)
                skey[pl.ds(cc * 16, 16)] = jnp.where(
                    neg, ~bu, bu | jnp.uint32(_HI))
                return carry

            lax.fori_loop(0, nchunk, mk, 0)

            def bit_step(it, carry):
                pt, pb = carry
                bitu = (31 - it).astype(jnp.uint32)
                ct = pt | lax.shift_left(jnp.uint32(1), bitu)
                cb = pb | lax.shift_left(jnp.uint32(1), bitu)
                cbinv = ~cb

                def cnt_chunk(cc, acc):
                    at, ab = acc
                    uk = skey[pl.ds(cc * 16, 16)]
                    at = at + jnp.where(uk >= ct, 1.0, 0.0)
                    ab = ab + jnp.where(uk <= cbinv, 1.0, 0.0)
                    return (at, ab)

                z = jnp.zeros((16,), jnp.float32)
                at, ab = lax.fori_loop(0, nchunk, cnt_chunk, (z, z))
                kf = jnp.float32(k)
                pt = jnp.where(_sum16(at) >= kf, ct, pt)
                pb = jnp.where(_sum16(ab) >= kf, cb, pb)
                return (pt, pb)

            pt, pb = lax.fori_loop(0, 32, bit_step,
                                   (jnp.uint32(0), jnp.uint32(0)))
            thr_b = ~pb

            def sums_chunk(cc, acc):
                sst, snt, cet, set_, ssb, snb, ceb, seb, cgt, cgb = acc
                uk = skey[pl.ds(cc * 16, 16)]
                sv = srow[pl.ds(cc * 16, 16)]
                nv = nrow[pl.ds(cc * 16, 16)]
                mt = uk > pt
                met = uk == pt
                mb = uk < thr_b
                meb = uk == thr_b
                sst = sst + jnp.where(mt, sv, 0.0)
                snt = snt + jnp.where(mt, nv, 0.0)
                cgt = cgt + jnp.where(mt, 1.0, 0.0)
                cet = cet + jnp.where(met, 1.0, 0.0)
                set_ = set_ + jnp.where(met, nv, 0.0)
                ssb = ssb + jnp.where(mb, sv, 0.0)
                snb = snb + jnp.where(mb, nv, 0.0)
                cgb = cgb + jnp.where(mb, 1.0, 0.0)
                ceb = ceb + jnp.where(meb, 1.0, 0.0)
                seb = seb + jnp.where(meb, nv, 0.0)
                return (sst, snt, cet, set_, ssb, snb, ceb, seb, cgt, cgb)

            zf = jnp.zeros((16,), jnp.float32)
            sst, snt, cet, set_, ssb, snb, ceb, seb, cgt, cgb = lax.fori_loop(
                0, nchunk, sums_chunk,
                (zf, zf, zf, zf, zf, zf, zf, zf, zf, zf))

            def dec(u_scalar):
                us = jnp.full((16,), u_scalar, jnp.uint32)
                pos = us >= jnp.full((16,), _HI, jnp.uint32)
                bits = jnp.where(pos, us ^ jnp.full((16,), _HI, jnp.uint32),
                                 ~us)
                return lax.bitcast_convert_type(bits, jnp.float32)

            vals12 = [
                _sum16(sst), _sum16(cgt), _sum16(snt),
                _sum16(cet), _sum16(set_), None,
                _sum16(ssb), _sum16(cgb), _sum16(snb),
                _sum16(ceb), _sum16(seb), None,
            ]
            lane = lax.iota(jnp.int32, 16)
            out_vec = jnp.zeros((16,), jnp.float32)
            for idx, sc in enumerate(vals12):
                if sc is not None:
                    out_vec = jnp.where(lane == idx, sc, out_vec)
            out_vec = jnp.where(lane == 5, dec(pt), out_vec)
            out_vec = jnp.where(lane == 11, dec(thr_b), out_vec)
            obuf[...] = out_vec
            pltpu.sync_copy(obuf, out_ref.at[r])


def _final_body(k, ncls, seg_ref, sums_ref, an_ref, bn_ref, as_ref, bs_ref,
                sm_ref):
    nb = an_ref.shape[0]
    a = sums_ref[...]

    def col(q):
        return a[:, q].reshape(nb, ncls)

    kf = jnp.float32(k)
    sst, cgt, snt, cet, set_, tht = (col(q) for q in range(6))
    ssb, cgb, snb, ceb, seb, thb = (col(q) for q in range(6, 12))

    def softmax_rows(v):
        m = jnp.max(v, axis=1, keepdims=True)
        e = jnp.exp(v - m)
        return e / jnp.sum(e, axis=1, keepdims=True)

    ties_t = kf - cgt
    an_ref[...] = ((snt + ties_t * set_ / cet) / kf).reshape(nb, 1, ncls)
    as_ref[...] = softmax_rows((sst + ties_t * tht) / kf).reshape(nb, 1, ncls)
    ties_b = kf - cgb
    bn_ref[...] = ((snb + ties_b * seb / ceb) / kf).reshape(nb, 1, ncls)
    bs_ref[...] = softmax_rows((ssb + ties_b * thb) / kf).reshape(nb, 1, ncls)

    s3 = seg_ref[...]
    m = jnp.max(s3, axis=2, keepdims=True)
    e = jnp.exp(s3 - m)
    sm_ref[...] = e / jnp.sum(e, axis=2, keepdims=True)


@jax.jit
def kernel(x, Wc, bc, proxy):
    B, T, D = x.shape
    C = proxy.shape[0]
    tile = 512 if T % 512 == 0 and T >= 512 else T
    nt = T // tile
    k = max(T // 8, 1)

    wt = jnp.transpose(Wc.astype(jnp.bfloat16), (2, 1, 0))
    pxt = jnp.transpose(proxy, (1, 0))
    bc2 = bc.reshape(1, D)

    zrow = jnp.zeros((B, 1, D), jnp.float32)
    last_rows = x[:, tile - 1::tile, :]
    first_rows = x[:, ::tile, :]
    edge_prev = jnp.concatenate(
        [zrow, last_rows[:, :nt - 1, :]], axis=1).reshape(B, nt, 1, D)
    edge_next = jnp.concatenate(
        [first_rows[:, 1:, :], zrow], axis=1).reshape(B, nt, 1, D)

    feat, seg, sct, nrm = pl.pallas_call(
        _conv_body,
        grid=(B, nt),
        in_specs=[
            pl.BlockSpec((1, tile, D), lambda b, i: (b, i, 0)),
            pl.BlockSpec((1, 1, 1, D), lambda b, i: (b, i, 0, 0)),
            pl.BlockSpec((1, 1, 1, D), lambda b, i: (b, i, 0, 0)),
            pl.BlockSpec((3, D, D), lambda b, i: (0, 0, 0)),
            pl.BlockSpec((1, D), lambda b, i: (0, 0)),
            pl.BlockSpec((D, C), lambda b, i: (0, 0)),
        ],
        out_specs=[
            pl.BlockSpec((1, tile, D), lambda b, i: (b, i, 0)),
            pl.BlockSpec((1, tile, C), lambda b, i: (b, i, 0)),
            pl.BlockSpec((1, C, tile), lambda b, i: (b, 0, i)),
            pl.BlockSpec((1, 1, tile), lambda b, i: (b, 0, i)),
        ],
        out_shape=[
            jax.ShapeDtypeStruct((B, T, D), jnp.float32),
            jax.ShapeDtypeStruct((B, T, C), jnp.float32),
            jax.ShapeDtypeStruct((B, C, T), jnp.float32),
            jax.ShapeDtypeStruct((B, 1, T), jnp.float32),
        ],
        compiler_params=pltpu.CompilerParams(
            dimension_semantics=("parallel", "arbitrary")),
    )(x, edge_prev, edge_next, wt, bc2, pxt)

    sct2 = sct.reshape(B * C, T)
    nrm2 = nrm.reshape(B, T)

    mesh = plsc.VectorSubcoreMesh(core_axis_name="c", subcore_axis_name="s")
    sums = pl.kernel(
        functools.partial(_sc_topk_body, k, B * C, C, T),
        out_type=jax.ShapeDtypeStruct((B * C, 16), jnp.float32),
        mesh=mesh,
        scratch_types=[
            pltpu.VMEM((T,), jnp.float32),
            pltpu.VMEM((T,), jnp.float32),
            pltpu.VMEM((T,), jnp.uint32),
            pltpu.VMEM((16,), jnp.float32),
        ],
    )(sct2, nrm2)

    act_norm, bkg_norm, act_score, bkg_score, seg_sm = pl.pallas_call(
        functools.partial(_final_body, k, C),
        out_shape=[
            jax.ShapeDtypeStruct((B, 1, C), jnp.float32),
            jax.ShapeDtypeStruct((B, 1, C), jnp.float32),
            jax.ShapeDtypeStruct((B, 1, C), jnp.float32),
            jax.ShapeDtypeStruct((B, 1, C), jnp.float32),
            jax.ShapeDtypeStruct((B, T, C), jnp.float32),
        ],
    )(seg, sums)

    return (act_norm.reshape(B, C), bkg_norm.reshape(B, C), feat,
            act_score.reshape(B, C), bkg_score.reshape(B, C), seg_sm)
